# baseline (speedup 1.0000x reference)
import functools

import jax
import jax.numpy as jnp
from jax import lax
from jax.experimental import pallas as pl
from jax.experimental.pallas import tpu as pltpu
from jax.experimental.pallas import tpu_sc as plsc

_CH = 128
_NBUF = 8
_NW = 32


def _sc_gather(flat_tab, idx2):
  nch, ch = idx2.shape
  assert ch == _CH
  rows_total = nch * ch
  e = flat_tab.shape[1]
  nch_w = nch // _NW
  ng = nch_w // _NBUF
  assert nch_w * _NW == nch and ng * _NBUF == nch_w

  mesh = plsc.VectorSubcoreMesh(core_axis_name="c", subcore_axis_name="s")

  @functools.partial(
      pl.kernel,
      mesh=mesh,
      out_type=jax.ShapeDtypeStruct((rows_total, e), jnp.float32),
      scratch_types=[
          pltpu.VMEM((nch_w, _CH), jnp.int32),
          pltpu.VMEM((_NBUF, _CH, e), jnp.float32),
          pltpu.SemaphoreType.DMA((_NBUF,)),
      ],
      compiler_params=pltpu.CompilerParams(use_tc_tiling_on_sc=False),
  )
  def gather_k(tab_hbm, idx_hbm, out_hbm, idx_v, rows_v, gsem):
    cid = lax.axis_index("c")
    sid = lax.axis_index("s")
    wid = sid * 2 + cid
    chunk0 = wid * nch_w
    row0 = chunk0 * _CH
    pltpu.sync_copy(idx_hbm.at[pl.ds(chunk0, nch_w)], idx_v)

    def gather_dma(ci, b):
      return pltpu.make_async_copy(
          tab_hbm.at[idx_v.at[ci]], rows_v.at[b], gsem.at[b])

    for b in range(_NBUF):
      gather_dma(b, b).start()

    def body(g, carry):
      for b in range(_NBUF):
        ci = g * _NBUF + b
        gather_dma(ci, b).wait()
        pltpu.sync_copy(rows_v.at[b],
                        out_hbm.at[pl.ds(row0 + ci * _CH, _CH)])

        @pl.when(g < ng - 1)
        def _():
          gather_dma(ci + _NBUF, b).start()
      return carry

    lax.fori_loop(0, ng, body, 0)

  return gather_k(flat_tab, idx2)


def _mlp(emb2d, xnum_p, w1e, w1n, b1r, w2p, b2r, w3p, b3r):
  b, d_emb = emb2d.shape
  bloc = 1024
  grid = (b // bloc,)

  def mlp_k(emb_ref, xn_ref, w1e_ref, w1n_ref, b1_ref, w2_ref, b2_ref,
            w3_ref, b3_ref, out_ref):
    x1 = jnp.dot(emb_ref[...], w1e_ref[...],
                 preferred_element_type=jnp.float32)
    x1 = x1 + jnp.dot(xn_ref[...], w1n_ref[...],
                      preferred_element_type=jnp.float32)
    h1 = jnp.maximum(x1 + b1_ref[...], 0.0)
    h2 = jnp.maximum(
        jnp.dot(h1, w2_ref[...], preferred_element_type=jnp.float32)
        + b2_ref[...], 0.0)
    out_ref[...] = (
        jnp.dot(h2, w3_ref[...], preferred_element_type=jnp.float32)
        + b3_ref[...])

  return pl.pallas_call(
      mlp_k,
      grid=grid,
      in_specs=[
          pl.BlockSpec((bloc, d_emb), lambda i: (i, 0)),
          pl.BlockSpec((bloc, 128), lambda i: (i, 0)),
          pl.BlockSpec(w1e.shape, lambda i: (0, 0)),
          pl.BlockSpec(w1n.shape, lambda i: (0, 0)),
          pl.BlockSpec(b1r.shape, lambda i: (0, 0)),
          pl.BlockSpec(w2p.shape, lambda i: (0, 0)),
          pl.BlockSpec(b2r.shape, lambda i: (0, 0)),
          pl.BlockSpec(w3p.shape, lambda i: (0, 0)),
          pl.BlockSpec(b3r.shape, lambda i: (0, 0)),
      ],
      out_specs=pl.BlockSpec((bloc, 128), lambda i: (i, 0)),
      out_shape=jax.ShapeDtypeStruct((b, 128), jnp.float32),
      compiler_params=pltpu.CompilerParams(
          dimension_semantics=("arbitrary",)),
  )(emb2d, xnum_p, w1e, w1n, b1r, w2p, b2r, w3p, b3r)


def kernel(x_cat, x_num, tables, W1, b1, W2, b2, W3, b3):
  bsz, f = x_cat.shape
  v, e = tables.shape[1], tables.shape[2]

  flat_tab = tables.reshape(f * v, e)
  idx = (x_cat.astype(jnp.int32)
         + (jnp.arange(f, dtype=jnp.int32) * v)[None, :])
  idx2 = idx.reshape(-1, _CH)

  rows = _sc_gather(flat_tab, idx2)
  emb2d = rows.reshape(bsz, f * e)

  d_emb = f * e
  n_num = x_num.shape[1]
  xnum_p = jnp.pad(x_num.astype(jnp.float32), ((0, 0), (0, 128 - n_num)))
  w1e = W1[:, :d_emb].T
  w1n = jnp.pad(W1[:, d_emb:].T, ((0, 128 - n_num), (0, 0)))
  b1r = b1[None, :]
  w2p = jnp.pad(W2.T, ((0, 0), (0, 128 - W2.shape[0])))
  b2r = jnp.pad(b2, (0, 128 - b2.shape[0]))[None, :]
  w3p = jnp.pad(W3.T, ((0, 128 - W3.shape[1]), (0, 127)))
  b3r = jnp.broadcast_to(b3, (128,))[None, :]

  out128 = _mlp(emb2d, xnum_p, w1e, w1n, b1r, w2p, b2r, w3p, b3r)
  return out128[:, :1]
